# baseline (speedup 1.0000x reference)
import functools

import jax
import jax.numpy as jnp
from jax import lax
from jax.experimental import pallas as pl
from jax.experimental.pallas import tpu as pltpu
from jax.experimental.pallas import tpu_sc as plsc

L = 200
K = 64
N = 16
D = 128

NOBS = L * N
NHID = L * N
NCAND = L * K
B = NOBS + NHID + NCAND

NC = 2
NS = 16
NW = NC * NS
BPW = B // NW

CH = 50
S = L // CH
OBS_CH = CH * N
CAND_CH = CH * K


def _sc_gather(table, idx):
    mesh = plsc.VectorSubcoreMesh(core_axis_name="c", subcore_axis_name="s")

    @functools.partial(
        pl.kernel,
        mesh=mesh,
        out_type=jax.ShapeDtypeStruct((B, D), jnp.float32),
        scratch_types=[
            pltpu.VMEM((BPW,), jnp.int32),
            pltpu.VMEM((BPW, D), jnp.float32),
            pltpu.SemaphoreType.DMA,
        ],
    )
    def gather_kernel(table_hbm, idx_hbm, out_hbm, idx_v, rows_v, sem):
        wid = lax.axis_index("s") * NC + lax.axis_index("c")
        base = wid * BPW
        pltpu.sync_copy(idx_hbm.at[pl.ds(base, BPW)], idx_v)
        pltpu.async_copy(table_hbm.at[idx_v], rows_v, sem).wait()
        pltpu.sync_copy(rows_v, out_hbm.at[pl.ds(base, BPW)])

    return gather_kernel(table, idx)


def _tc_body(wel_ref, wtl_ref, wec_ref, wtc_ref, bec_ref, bel_ref,
             btc_ref, btl_ref, obs_ref, hid_ref, cand_ref, out_ref,
             fa_s, fb_s, ma_s, mb_s, mc_s, tp_s, tq_s, ee_s, m_s, mt_s):
    k = pl.program_id(0)

    def taps(v, w_ref):
        z = jnp.zeros((1, 1), jnp.float32)
        left = jnp.concatenate([v[:, 1:], z], axis=1)
        right = jnp.concatenate([z, v[:, :D - 1]], axis=1)
        g0 = w_ref[0] * left + w_ref[1] * v + w_ref[2] * right
        g1 = w_ref[3] * left + w_ref[4] * v + w_ref[5] * right
        return g0, g1

    def ctap(v, w_ref, i):
        zc = jnp.zeros((1, 1), jnp.float32)
        left = jnp.concatenate([v[1:, :], zc], axis=0)
        right = jnp.concatenate([zc, v[:D - 1, :]], axis=0)
        return w_ref[i] * left + w_ref[i + 1] * v + w_ref[i + 2] * right

    wl = wel_ref[...]
    tl = wtl_ref[...]
    gE0, gE1 = taps(wl, wec_ref)
    gT0, gT1 = taps(tl, wtc_ref)
    gT1c = ctap(tl.reshape(D, 1), wtc_ref, 3)
    gE1c = ctap(wl.reshape(D, 1), wec_ref, 3)
    gc = jnp.concatenate(
        [gT1c, gE1c, jnp.zeros((D, 6), jnp.float32)], axis=1)
    cE = bel_ref[0] + bec_ref[0] * jnp.sum(wl, axis=1, keepdims=True)
    cT = btl_ref[0] + btc_ref[0] * jnp.sum(tl, axis=1, keepdims=True)

    @pl.when(k < S)
    def _chunk():
        f = jnp.mean(obs_ref[...].reshape(CH, N, D), axis=1)
        m = jnp.mean(hid_ref[...].reshape(CH, N, D), axis=1)
        c3 = cand_ref[...].reshape(CH, K, D)
        r = pl.ds(k * CH, CH)
        fa = jnp.sum(f * gE0, axis=1, keepdims=True)
        fa_s[r, :] = fa
        fb_s[r, :] = jnp.sum(f * gE1, axis=1, keepdims=True)
        ma_s[r, :] = jnp.sum(m * gE0, axis=1, keepdims=True)
        mb_s[r, :] = jnp.sum(m * gT0, axis=1, keepdims=True)
        mc_s[r, :] = jnp.sum(m * gT1, axis=1, keepdims=True)
        tp = jnp.sum(c3 * gT0[None], axis=2)
        p8 = jnp.dot(cand_ref[...], gc,
                     preferred_element_type=jnp.float32)
        p83 = p8.reshape(CH, K, 8)
        tqc = p83[:, :, 0:1]
        bec = p83[:, :, 1:2]
        tp_s[r, :] = tp
        tq_s[r, :] = tqc
        fa3 = fa.reshape(CH, 1, 1)
        eec = jnp.exp(jax.nn.sigmoid(cE.reshape(1, 1, 1) + fa3 + bec))
        ee_s[r, :] = eec

        cT3 = cT.reshape(1, 1, 1)

        @pl.when(k == 0)
        def _m_first():
            sg = jax.nn.sigmoid(cT3 + tqc[1:CH] + tp[0:CH - 1, None, :])
            mm = jnp.exp(sg) * eec[1:CH]
            mt_s[0:CH - 1] = mm
            m_s[0:CH - 1] = jnp.swapaxes(mm, 1, 2)

        @pl.when(k > 0)
        def _m_rest():
            lo = k * CH - 1
            tpm = tp_s[pl.ds(lo, CH), :]
            tqm = tq_s[pl.ds(k * CH, CH)]
            eem = ee_s[pl.ds(k * CH, CH)]
            sg = jax.nn.sigmoid(cT3 + tqm + tpm[:, None, :])
            mm = jnp.exp(sg) * eem
            mt_s[pl.ds(lo, CH)] = mm
            m_s[pl.ds(lo, CH)] = jnp.swapaxes(mm, 1, 2)

    @pl.when(k == S - 1)
    def _finale():
        e_terms = jax.nn.sigmoid(cE + ma_s[...] + fb_s[...])
        t_terms = jax.nn.sigmoid(cT + mb_s[1:L, :] + mc_s[0:L - 1, :])
        gold = (jnp.sum(e_terms, axis=0, keepdims=True)
                + jnp.sum(t_terms, axis=0, keepdims=True))

        row = lax.broadcasted_iota(jnp.int32, (K, K), 0)
        col = lax.broadcasted_iota(jnp.int32, (K, K), 1)
        eye = jnp.where(row == col, 1.0, 0.0).astype(jnp.float32)
        m_s[pl.ds(L - 1, 1)] = eye.reshape(1, K, K)
        mt_s[pl.ds(L - 1, 1)] = eye.reshape(1, K, K)

        u0 = ee_s[0:1].reshape(K, 1)
        v0 = jnp.ones((K, 1), jnp.float32)

        def body(b, carry):
            u, v, lu, lv = carry
            for j in range(4):
                fi = (4 * b + j + L - 1) % L
                bi = L - 1 - 4 * b - j
                if j % 2 == 0:
                    u = jnp.sum(m_s[pl.ds(fi, 1)].reshape(K, K) * u,
                                axis=0, keepdims=True)
                    v = jnp.sum(mt_s[pl.ds(bi, 1)].reshape(K, K) * v,
                                axis=0, keepdims=True)
                else:
                    u = jnp.sum(mt_s[pl.ds(fi, 1)].reshape(K, K) * u,
                                axis=1, keepdims=True)
                    v = jnp.sum(m_s[pl.ds(bi, 1)].reshape(K, K) * v,
                                axis=1, keepdims=True)
            su = jnp.max(u, axis=0, keepdims=True)
            sv = jnp.max(v, axis=0, keepdims=True)
            return u / su, v / sv, lu + jnp.log(su), lv + jnp.log(sv)

        z11 = jnp.zeros((1, 1), jnp.float32)
        u, v, lu, lv = lax.fori_loop(0, L // 8, body, (u0, v0, z11, z11))
        z = jnp.sum(m_s[pl.ds(99, 1)].reshape(K, K) * u,
                    axis=0, keepdims=True)
        tot = jnp.sum(z * v.reshape(1, K), axis=1, keepdims=True)
        fwd = jnp.log(tot) + lu + lv
        out_ref[...] = jnp.broadcast_to(fwd - gold, (1, 128))


def _tc_crf(rows, w_el, w_tl, w_ec6, w_tc6, b_ec, b_el, b_tc, b_tl):
    hid_b0 = NOBS // OBS_CH
    cand_b0 = (NOBS + NHID) // CAND_CH

    smem = pl.BlockSpec(memory_space=pltpu.SMEM)
    return pl.pallas_call(
        _tc_body,
        grid=(S,),
        in_specs=[
            pl.BlockSpec((1, 128), lambda k: (0, 0)),
            pl.BlockSpec((1, 128), lambda k: (0, 0)),
            smem, smem, smem, smem, smem, smem,
            pl.BlockSpec((OBS_CH, D), lambda k: (k, 0)),
            pl.BlockSpec((OBS_CH, D), lambda k: (hid_b0 + k, 0)),
            pl.BlockSpec((CAND_CH, D), lambda k: (cand_b0 + k, 0)),
        ],
        out_specs=pl.BlockSpec((1, 128), lambda k: (0, 0)),
        out_shape=jax.ShapeDtypeStruct((1, 128), jnp.float32),
        scratch_shapes=[
            pltpu.VMEM((L, 1), jnp.float32),
            pltpu.VMEM((L, 1), jnp.float32),
            pltpu.VMEM((L, 1), jnp.float32),
            pltpu.VMEM((L, 1), jnp.float32),
            pltpu.VMEM((L, 1), jnp.float32),
            pltpu.VMEM((L, K), jnp.float32),
            pltpu.VMEM((L, K, 1), jnp.float32),
            pltpu.VMEM((L, K, 1), jnp.float32),
            pltpu.VMEM((L, K, K), jnp.float32),
            pltpu.VMEM((L, K, K), jnp.float32),
        ],
        compiler_params=pltpu.CompilerParams(
            dimension_semantics=("arbitrary",)),
    )(w_el, w_tl, w_ec6, w_tc6, b_ec, b_el, b_tc, b_tl, rows, rows, rows)


def kernel(W_embed, w_ec, b_ec, w_el, b_el, w_tc, b_tc, w_tl, b_tl,
           observed, candidates, hidden_states):
    idx = jnp.concatenate([
        observed.reshape(-1), hidden_states.reshape(-1),
        candidates.reshape(-1)]).astype(jnp.int32)
    rows = _sc_gather(W_embed, idx)
    out = _tc_crf(rows, w_el, w_tl, w_ec.reshape(6), w_tc.reshape(6),
                  b_ec, b_el, b_tc, b_tl)
    return out[0, 0:1]


# speedup vs baseline: 1.0285x; 1.0285x over previous
import functools

import jax
import jax.numpy as jnp
from jax import lax
from jax.experimental import pallas as pl
from jax.experimental.pallas import tpu as pltpu
from jax.experimental.pallas import tpu_sc as plsc

L = 200
K = 64
N = 16
D = 128

NOBS = L * N
NHID = L * N
NCAND = L * K
B = NOBS + NHID + NCAND

NC = 2
NS = 16
NW = NC * NS
BPW = B // NW

CH = 50
S = L // CH
OBS_CH = CH * N
CAND_CH = CH * K


def _sc_gather(table, idx):
    mesh = plsc.VectorSubcoreMesh(core_axis_name="c", subcore_axis_name="s")

    @functools.partial(
        pl.kernel,
        mesh=mesh,
        out_type=jax.ShapeDtypeStruct((B, D), jnp.float32),
        scratch_types=[
            pltpu.VMEM((BPW,), jnp.int32),
            pltpu.VMEM((BPW, D), jnp.float32),
            pltpu.SemaphoreType.DMA,
        ],
    )
    def gather_kernel(table_hbm, idx_hbm, out_hbm, idx_v, rows_v, sem):
        wid = lax.axis_index("s") * NC + lax.axis_index("c")
        base = wid * BPW
        pltpu.sync_copy(idx_hbm.at[pl.ds(base, BPW)], idx_v)
        pltpu.async_copy(table_hbm.at[idx_v], rows_v, sem).wait()
        pltpu.sync_copy(rows_v, out_hbm.at[pl.ds(base, BPW)])

    return gather_kernel(table, idx)


def _tc_body(wel_ref, wtl_ref, wec_ref, wtc_ref, bec_ref, bel_ref,
             btc_ref, btl_ref, obs_ref, hid_ref, cand_ref, out_ref,
             fa_s, fb_s, ma_s, mb_s, mc_s, tp_s, tq_s, ee_s, m_s, mt_s):
    k = pl.program_id(0)

    def taps(v, w_ref):
        z = jnp.zeros((1, 1), jnp.float32)
        left = jnp.concatenate([v[:, 1:], z], axis=1)
        right = jnp.concatenate([z, v[:, :D - 1]], axis=1)
        g0 = w_ref[0] * left + w_ref[1] * v + w_ref[2] * right
        g1 = w_ref[3] * left + w_ref[4] * v + w_ref[5] * right
        return g0, g1

    wl = wel_ref[...]
    tl = wtl_ref[...]
    gE0, gE1 = taps(wl, wec_ref)
    gT0, gT1 = taps(tl, wtc_ref)
    cE = bel_ref[0] + bec_ref[0] * jnp.sum(wl, axis=1, keepdims=True)
    cT = btl_ref[0] + btc_ref[0] * jnp.sum(tl, axis=1, keepdims=True)

    @pl.when(k < S)
    def _chunk():
        f = jnp.mean(obs_ref[...].reshape(CH, N, D), axis=1)
        m = jnp.mean(hid_ref[...].reshape(CH, N, D), axis=1)
        c3 = cand_ref[...].reshape(CH, K, D)
        r = pl.ds(k * CH, CH)
        fa = jnp.sum(f * gE0, axis=1, keepdims=True)
        fa_s[r, :] = fa
        fb_s[r, :] = jnp.sum(f * gE1, axis=1, keepdims=True)
        ma_s[r, :] = jnp.sum(m * gE0, axis=1, keepdims=True)
        mb_s[r, :] = jnp.sum(m * gT0, axis=1, keepdims=True)
        mc_s[r, :] = jnp.sum(m * gT1, axis=1, keepdims=True)
        tp = jnp.sum(c3 * gT0[None], axis=2)
        tq = jnp.sum(c3 * gT1[None], axis=2)
        be = jnp.sum(c3 * gE1[None], axis=2)
        tp_s[r, :] = tp
        tq_s[r, :] = tq
        ee = jnp.exp(jax.nn.sigmoid(cE + fa + be))
        ee_s[r, :] = ee

        cT3 = cT.reshape(1, 1, 1)

        @pl.when(k == 0)
        def _m_first():
            sg = jax.nn.sigmoid(cT3 + tp[0:CH - 1, :, None]
                                + tq[1:CH, None, :])
            mm = jnp.exp(sg) * ee[1:CH][:, None, :]
            m_s[0:CH - 1] = mm
            mt_s[0:CH - 1] = jnp.swapaxes(mm, 1, 2)

        @pl.when(k > 0)
        def _m_rest():
            lo = k * CH - 1
            tpm = tp_s[pl.ds(lo, CH), :]
            tqm = tq_s[pl.ds(k * CH, CH), :]
            eem = ee_s[pl.ds(k * CH, CH), :]
            sg = jax.nn.sigmoid(cT3 + tpm[:, :, None] + tqm[:, None, :])
            mm = jnp.exp(sg) * eem[:, None, :]
            m_s[pl.ds(lo, CH)] = mm
            mt_s[pl.ds(lo, CH)] = jnp.swapaxes(mm, 1, 2)

    @pl.when(k == S - 1)
    def _finale():
        e_terms = jax.nn.sigmoid(cE + ma_s[...] + fb_s[...])
        t_terms = jax.nn.sigmoid(cT + mb_s[1:L, :] + mc_s[0:L - 1, :])
        gold = (jnp.sum(e_terms, axis=0, keepdims=True)
                + jnp.sum(t_terms, axis=0, keepdims=True))

        row = lax.broadcasted_iota(jnp.int32, (K, K), 0)
        col = lax.broadcasted_iota(jnp.int32, (K, K), 1)
        eye = jnp.where(row == col, 1.0, 0.0).astype(jnp.float32)
        m_s[pl.ds(L - 1, 1)] = eye.reshape(1, K, K)
        mt_s[pl.ds(L - 1, 1)] = eye.reshape(1, K, K)

        u0 = ee_s[0:1, :]
        v0 = jnp.ones((1, K), jnp.float32)

        def body(b, carry):
            u, v, lu, lv = carry
            for j in range(4):
                fi = (4 * b + j + L - 1) % L
                bi = L - 1 - 4 * b - j
                if j % 2 == 0:
                    u = jnp.sum(mt_s[pl.ds(fi, 1)].reshape(K, K) * u,
                                axis=1, keepdims=True)
                    v = jnp.sum(m_s[pl.ds(bi, 1)].reshape(K, K) * v,
                                axis=1, keepdims=True)
                else:
                    u = jnp.sum(m_s[pl.ds(fi, 1)].reshape(K, K) * u,
                                axis=0, keepdims=True)
                    v = jnp.sum(mt_s[pl.ds(bi, 1)].reshape(K, K) * v,
                                axis=0, keepdims=True)
            su = jnp.max(u, axis=1, keepdims=True)
            sv = jnp.max(v, axis=1, keepdims=True)
            return u / su, v / sv, lu + jnp.log(su), lv + jnp.log(sv)

        z11 = jnp.zeros((1, 1), jnp.float32)
        u, v, lu, lv = lax.fori_loop(0, L // 8, body, (u0, v0, z11, z11))
        w = jnp.sum(m_s[pl.ds(99, 1)].reshape(K, K) * v,
                    axis=1, keepdims=True)
        tot = jnp.sum(u * w.reshape(1, K), axis=1, keepdims=True)
        fwd = jnp.log(tot) + lu + lv
        out_ref[...] = jnp.broadcast_to(fwd - gold, (1, 128))


def _tc_crf(rows, w_el, w_tl, w_ec6, w_tc6, b_ec, b_el, b_tc, b_tl):
    hid_b0 = NOBS // OBS_CH
    cand_b0 = (NOBS + NHID) // CAND_CH

    smem = pl.BlockSpec(memory_space=pltpu.SMEM)
    return pl.pallas_call(
        _tc_body,
        grid=(S,),
        in_specs=[
            pl.BlockSpec((1, 128), lambda k: (0, 0)),
            pl.BlockSpec((1, 128), lambda k: (0, 0)),
            smem, smem, smem, smem, smem, smem,
            pl.BlockSpec((OBS_CH, D), lambda k: (k, 0)),
            pl.BlockSpec((OBS_CH, D), lambda k: (hid_b0 + k, 0)),
            pl.BlockSpec((CAND_CH, D), lambda k: (cand_b0 + k, 0)),
        ],
        out_specs=pl.BlockSpec((1, 128), lambda k: (0, 0)),
        out_shape=jax.ShapeDtypeStruct((1, 128), jnp.float32),
        scratch_shapes=[
            pltpu.VMEM((L, 1), jnp.float32),
            pltpu.VMEM((L, 1), jnp.float32),
            pltpu.VMEM((L, 1), jnp.float32),
            pltpu.VMEM((L, 1), jnp.float32),
            pltpu.VMEM((L, 1), jnp.float32),
            pltpu.VMEM((L, K), jnp.float32),
            pltpu.VMEM((L, K), jnp.float32),
            pltpu.VMEM((L, K), jnp.float32),
            pltpu.VMEM((L, K, K), jnp.float32),
            pltpu.VMEM((L, K, K), jnp.float32),
        ],
        compiler_params=pltpu.CompilerParams(
            dimension_semantics=("arbitrary",)),
    )(w_el, w_tl, w_ec6, w_tc6, b_ec, b_el, b_tc, b_tl, rows, rows, rows)


def kernel(W_embed, w_ec, b_ec, w_el, b_el, w_tc, b_tc, w_tl, b_tl,
           observed, candidates, hidden_states):
    idx = jnp.concatenate([
        observed.reshape(-1), hidden_states.reshape(-1),
        candidates.reshape(-1)]).astype(jnp.int32)
    rows = _sc_gather(W_embed, idx)
    out = _tc_crf(rows, w_el, w_tl, w_ec.reshape(6), w_tc.reshape(6),
                  b_ec, b_el, b_tc, b_tl)
    return out[0, 0:1]


# speedup vs baseline: 1.0553x; 1.0261x over previous
import functools

import jax
import jax.numpy as jnp
from jax import lax
from jax.experimental import pallas as pl
from jax.experimental.pallas import tpu as pltpu
from jax.experimental.pallas import tpu_sc as plsc

L = 200
K = 64
N = 16
D = 128

NOBS = L * N
NHID = L * N
NCAND = L * K
B = NOBS + NHID + NCAND

NC = 2
NS = 16
NW = NC * NS
BPW = B // NW

CH = 50
S = L // CH
OBS_CH = CH * N
CAND_CH = CH * K


def _sc_gather(table, idx):
    mesh = plsc.VectorSubcoreMesh(core_axis_name="c", subcore_axis_name="s")

    @functools.partial(
        pl.kernel,
        mesh=mesh,
        out_type=jax.ShapeDtypeStruct((B, D), jnp.float32),
        scratch_types=[
            pltpu.VMEM((BPW,), jnp.int32),
            pltpu.VMEM((BPW, D), jnp.float32),
            pltpu.SemaphoreType.DMA,
        ],
    )
    def gather_kernel(table_hbm, idx_hbm, out_hbm, idx_v, rows_v, sem):
        wid = lax.axis_index("s") * NC + lax.axis_index("c")
        base = wid * BPW
        pltpu.sync_copy(idx_hbm.at[pl.ds(base, BPW)], idx_v)
        pltpu.async_copy(table_hbm.at[idx_v], rows_v, sem).wait()
        pltpu.sync_copy(rows_v, out_hbm.at[pl.ds(base, BPW)])

    return gather_kernel(table, idx)


def _tc_body(wel_ref, wtl_ref, wec_ref, wtc_ref, bec_ref, bel_ref,
             btc_ref, btl_ref, obs_ref, hid_ref, cand_ref, out_ref,
             fa_s, fb_s, ma_s, mb_s, mc_s, tp_s, tq_s, ee_s, m_s, mt_s):
    k = pl.program_id(0)

    def taps(v, w_ref):
        z = jnp.zeros((1, 1), jnp.float32)
        left = jnp.concatenate([v[:, 1:], z], axis=1)
        right = jnp.concatenate([z, v[:, :D - 1]], axis=1)
        g0 = w_ref[0] * left + w_ref[1] * v + w_ref[2] * right
        g1 = w_ref[3] * left + w_ref[4] * v + w_ref[5] * right
        return g0, g1

    wl = wel_ref[...]
    tl = wtl_ref[...]
    gE0, gE1 = taps(wl, wec_ref)
    gT0, gT1 = taps(tl, wtc_ref)
    cE = bel_ref[0] + bec_ref[0] * jnp.sum(wl, axis=1, keepdims=True)
    cT = btl_ref[0] + btc_ref[0] * jnp.sum(tl, axis=1, keepdims=True)

    @pl.when(k < S)
    def _chunk():
        f = jnp.mean(obs_ref[...].reshape(CH, N, D), axis=1)
        m = jnp.mean(hid_ref[...].reshape(CH, N, D), axis=1)
        c3 = cand_ref[...].reshape(CH, K, D)
        r = pl.ds(k * CH, CH)
        fa = jnp.sum(f * gE0, axis=1, keepdims=True)
        fa_s[r, :] = fa
        fb_s[r, :] = jnp.sum(f * gE1, axis=1, keepdims=True)
        ma_s[r, :] = jnp.sum(m * gE0, axis=1, keepdims=True)
        mb_s[r, :] = jnp.sum(m * gT0, axis=1, keepdims=True)
        mc_s[r, :] = jnp.sum(m * gT1, axis=1, keepdims=True)
        tp = jnp.sum(c3 * gT0[None], axis=2)
        tq = jnp.sum(c3 * gT1[None], axis=2)
        be = jnp.sum(c3 * gE1[None], axis=2)
        tp_s[r, :] = tp
        tq_s[r, :] = tq
        ee = jnp.exp(jax.nn.sigmoid(cE + fa + be))
        ee_s[r, :] = ee

        cT3 = cT.reshape(1, 1, 1)

        @pl.when(k == 0)
        def _m_first():
            sg = jax.nn.sigmoid(cT3 + tp[0:CH - 1, :, None]
                                + tq[1:CH, None, :])
            mm = jnp.exp(sg) * ee[1:CH][:, None, :]
            m_s[0:CH - 1] = mm
            mt_s[0:CH - 1] = jnp.swapaxes(mm, 1, 2)

        @pl.when(k > 0)
        def _m_rest():
            lo = k * CH - 1
            tpm = tp_s[pl.ds(lo, CH), :]
            tqm = tq_s[pl.ds(k * CH, CH), :]
            eem = ee_s[pl.ds(k * CH, CH), :]
            sg = jax.nn.sigmoid(cT3 + tpm[:, :, None] + tqm[:, None, :])
            mm = jnp.exp(sg) * eem[:, None, :]
            m_s[pl.ds(lo, CH)] = mm
            mt_s[pl.ds(lo, CH)] = jnp.swapaxes(mm, 1, 2)

    @pl.when(k == S - 1)
    def _finale():
        e_terms = jax.nn.sigmoid(cE + ma_s[...] + fb_s[...])
        t_terms = jax.nn.sigmoid(cT + mb_s[1:L, :] + mc_s[0:L - 1, :])
        gold = (jnp.sum(e_terms, axis=0, keepdims=True)
                + jnp.sum(t_terms, axis=0, keepdims=True))

        row = lax.broadcasted_iota(jnp.int32, (K, K), 0)
        col = lax.broadcasted_iota(jnp.int32, (K, K), 1)
        eye = jnp.where(row == col, 1.0, 0.0).astype(jnp.float32)
        m_s[pl.ds(L - 1, 1)] = eye.reshape(1, K, K)
        mt_s[pl.ds(L - 1, 1)] = eye.reshape(1, K, K)

        u0 = ee_s[0:1, :]
        v0 = jnp.ones((1, K), jnp.float32)

        def body(b, carry):
            u, v, lu, lv = carry
            for j in range(10):
                fi = (10 * b + j + L - 1) % L
                bi = L - 1 - 10 * b - j
                if j % 2 == 0:
                    u = jnp.sum(mt_s[pl.ds(fi, 1)].reshape(K, K) * u,
                                axis=1, keepdims=True)
                    v = jnp.sum(m_s[pl.ds(bi, 1)].reshape(K, K) * v,
                                axis=1, keepdims=True)
                else:
                    u = jnp.sum(m_s[pl.ds(fi, 1)].reshape(K, K) * u,
                                axis=0, keepdims=True)
                    v = jnp.sum(mt_s[pl.ds(bi, 1)].reshape(K, K) * v,
                                axis=0, keepdims=True)
            su = jnp.max(u, axis=1, keepdims=True)
            sv = jnp.max(v, axis=1, keepdims=True)
            return u / su, v / sv, lu + jnp.log(su), lv + jnp.log(sv)

        z11 = jnp.zeros((1, 1), jnp.float32)
        u, v, lu, lv = lax.fori_loop(0, L // 20, body, (u0, v0, z11, z11))
        w = jnp.sum(m_s[pl.ds(99, 1)].reshape(K, K) * v,
                    axis=1, keepdims=True)
        tot = jnp.sum(u * w.reshape(1, K), axis=1, keepdims=True)
        fwd = jnp.log(tot) + lu + lv
        out_ref[...] = jnp.broadcast_to(fwd - gold, (1, 128))


def _tc_crf(rows, w_el, w_tl, w_ec6, w_tc6, b_ec, b_el, b_tc, b_tl):
    hid_b0 = NOBS // OBS_CH
    cand_b0 = (NOBS + NHID) // CAND_CH

    smem = pl.BlockSpec(memory_space=pltpu.SMEM)
    return pl.pallas_call(
        _tc_body,
        grid=(S,),
        in_specs=[
            pl.BlockSpec((1, 128), lambda k: (0, 0)),
            pl.BlockSpec((1, 128), lambda k: (0, 0)),
            smem, smem, smem, smem, smem, smem,
            pl.BlockSpec((OBS_CH, D), lambda k: (k, 0)),
            pl.BlockSpec((OBS_CH, D), lambda k: (hid_b0 + k, 0)),
            pl.BlockSpec((CAND_CH, D), lambda k: (cand_b0 + k, 0)),
        ],
        out_specs=pl.BlockSpec((1, 128), lambda k: (0, 0)),
        out_shape=jax.ShapeDtypeStruct((1, 128), jnp.float32),
        scratch_shapes=[
            pltpu.VMEM((L, 1), jnp.float32),
            pltpu.VMEM((L, 1), jnp.float32),
            pltpu.VMEM((L, 1), jnp.float32),
            pltpu.VMEM((L, 1), jnp.float32),
            pltpu.VMEM((L, 1), jnp.float32),
            pltpu.VMEM((L, K), jnp.float32),
            pltpu.VMEM((L, K), jnp.float32),
            pltpu.VMEM((L, K), jnp.float32),
            pltpu.VMEM((L, K, K), jnp.float32),
            pltpu.VMEM((L, K, K), jnp.float32),
        ],
        compiler_params=pltpu.CompilerParams(
            dimension_semantics=("arbitrary",)),
    )(w_el, w_tl, w_ec6, w_tc6, b_ec, b_el, b_tc, b_tl, rows, rows, rows)


def kernel(W_embed, w_ec, b_ec, w_el, b_el, w_tc, b_tc, w_tl, b_tl,
           observed, candidates, hidden_states):
    idx = jnp.concatenate([
        observed.reshape(-1), hidden_states.reshape(-1),
        candidates.reshape(-1)]).astype(jnp.int32)
    rows = _sc_gather(W_embed, idx)
    out = _tc_crf(rows, w_el, w_tl, w_ec.reshape(6), w_tc.reshape(6),
                  b_ec, b_el, b_tc, b_tl)
    return out[0, 0:1]
